# Initial kernel scaffold; baseline (speedup 1.0000x reference)
import jax
import jax.numpy as jnp
from jax.experimental import pallas as pl


def kernel(x, adj, W, b):
    raise NotImplementedError("write your pallas kernel here")


# speedup vs baseline: 2.2921x; 2.2921x over previous
import jax
import jax.numpy as jnp
from jax.experimental import pallas as pl

B, N, D = 8, 2048, 256


def _sgconv_block(x_ref, adj_ref, w_ref, b_ref, out_ref):
    adj = adj_ref[0]
    deg = jnp.sum(jnp.abs(adj), axis=1)
    norm = jax.lax.rsqrt(deg + 1e-6)[:, None]
    support = jnp.dot(x_ref[0], w_ref[...], preferred_element_type=jnp.float32)
    tmp = support * norm
    out = jnp.dot(adj, tmp, preferred_element_type=jnp.float32)
    out_ref[0] = jnp.maximum(out * norm + b_ref[...], 0.0)


def kernel(x, adj, W, b):
    b2d = b.reshape(1, D)
    return pl.pallas_call(
        _sgconv_block,
        grid=(B,),
        in_specs=[
            pl.BlockSpec((1, N, D), lambda i: (i, 0, 0)),
            pl.BlockSpec((1, N, N), lambda i: (i, 0, 0)),
            pl.BlockSpec((D, D), lambda i: (0, 0)),
            pl.BlockSpec((1, D), lambda i: (0, 0)),
        ],
        out_specs=pl.BlockSpec((1, N, D), lambda i: (i, 0, 0)),
        out_shape=jax.ShapeDtypeStruct((B, N, D), jnp.float32),
    )(x, adj, W, b2d)
